# baseline (speedup 1.0000x reference)
import functools

import jax
import jax.numpy as jnp
from jax import lax
from jax.experimental import pallas as pl
from jax.experimental.pallas import tpu as pltpu
from jax.experimental.pallas import tpu_sc as plsc

_NUM_EMBEDDINGS = 1000000
_DIM = 64
_BATCH = 16384
_HIST = 200
_B = _BATCH * _HIST

_NC = 2
_NS = 16
_NW = _NC * _NS

_B_PER_W = _B // _NW
_CHUNK = 1024
_STEPS = _B_PER_W // _CHUNK


def _body(table_hbm, idx_hbm, out_hbm, idx_v, rows_v, sem):
    wid = lax.axis_index("s") * _NC + lax.axis_index("c")
    base = wid * _B_PER_W

    def step(i, carry):
        off = base + i * _CHUNK
        pltpu.sync_copy(idx_hbm.at[pl.ds(off, _CHUNK)], idx_v)
        pltpu.async_copy(table_hbm.at[idx_v], rows_v, sem).wait()
        pltpu.sync_copy(rows_v, out_hbm.at[pl.ds(off, _CHUNK)])
        return carry

    lax.fori_loop(0, _STEPS, step, 0)


_gather = functools.partial(
    pl.kernel,
    out_type=jax.ShapeDtypeStruct((_B, _DIM), jnp.float32),
    mesh=plsc.VectorSubcoreMesh(core_axis_name="c", subcore_axis_name="s"),
    scratch_types=[
        pltpu.VMEM((_CHUNK,), jnp.int32),
        pltpu.VMEM((_CHUNK, _DIM), jnp.float32),
        pltpu.SemaphoreType.DMA,
    ],
    compiler_params=pltpu.CompilerParams(use_tc_tiling_on_sc=False),
)(_body)


@jax.jit
def kernel(token_ids, weights):
    flat = token_ids.reshape(_B)
    rows = _gather(weights, flat)
    return rows.reshape(_BATCH, _HIST, _DIM)


# speedup vs baseline: 1.0336x; 1.0336x over previous
import functools

import jax
import jax.numpy as jnp
from jax import lax
from jax.experimental import pallas as pl
from jax.experimental.pallas import tpu as pltpu
from jax.experimental.pallas import tpu_sc as plsc

_NUM_EMBEDDINGS = 1000000
_DIM = 64
_BATCH = 16384
_HIST = 200
_B = _BATCH * _HIST

_NC = 2
_NS = 16
_NW = _NC * _NS

_B_PER_W = _B // _NW
_CHUNK = 800
_STEPS = _B_PER_W // _CHUNK
_G = _STEPS // 2


def _body(table_hbm, idx_hbm, out_hbm, idx_v, rows_v,
          isem0, isem1, gsem0, gsem1, osem0, osem1):
    wid = lax.axis_index("s") * _NC + lax.axis_index("c")
    base = wid * _B_PER_W
    isems = (isem0, isem1)
    gsems = (gsem0, gsem1)
    osems = (osem0, osem1)

    def start_idx(i, b):
        off = base + i * _CHUNK
        pltpu.async_copy(idx_hbm.at[pl.ds(off, _CHUNK)], idx_v.at[b], isems[b])

    def wait_idx(b):
        pltpu.make_async_copy(idx_hbm.at[pl.ds(0, _CHUNK)], idx_v.at[b],
                              isems[b]).wait()

    def start_gather(b):
        pltpu.async_copy(table_hbm.at[idx_v.at[b]], rows_v.at[b], gsems[b])

    def wait_gather(b):
        pltpu.make_async_copy(table_hbm.at[idx_v.at[b]], rows_v.at[b],
                              gsems[b]).wait()

    def start_write(i, b):
        off = base + i * _CHUNK
        pltpu.async_copy(rows_v.at[b], out_hbm.at[pl.ds(off, _CHUNK)], osems[b])

    def wait_write(b):
        pltpu.make_async_copy(rows_v.at[b], out_hbm.at[pl.ds(0, _CHUNK)],
                              osems[b]).wait()

    def chunk_step(i, b):
        nb = 1 - b

        @pl.when(i > 0)
        def _():
            wait_gather(nb)
            start_write(i - 1, nb)

        @pl.when(i + 1 < _STEPS)
        def _():
            start_idx(i + 1, nb)

        wait_idx(b)

        @pl.when(i > 1)
        def _():
            wait_write(b)

        start_gather(b)

    start_idx(0, 0)

    def gstep(g, carry):
        i0 = 2 * g
        chunk_step(i0, 0)
        chunk_step(i0 + 1, 1)
        return carry

    lax.fori_loop(0, _G, gstep, 0)

    last = _STEPS - 1
    wait_gather(1)
    start_write(last, 1)
    wait_write(0)
    wait_write(1)


_gather = functools.partial(
    pl.kernel,
    out_type=jax.ShapeDtypeStruct((_B, _DIM), jnp.float32),
    mesh=plsc.VectorSubcoreMesh(core_axis_name="c", subcore_axis_name="s"),
    scratch_types=[
        pltpu.VMEM((2, _CHUNK), jnp.int32),
        pltpu.VMEM((2, _CHUNK, _DIM), jnp.float32),
        pltpu.SemaphoreType.DMA,
        pltpu.SemaphoreType.DMA,
        pltpu.SemaphoreType.DMA,
        pltpu.SemaphoreType.DMA,
        pltpu.SemaphoreType.DMA,
        pltpu.SemaphoreType.DMA,
    ],
    compiler_params=pltpu.CompilerParams(use_tc_tiling_on_sc=False),
)(_body)


@jax.jit
def kernel(token_ids, weights):
    flat = token_ids.reshape(_B)
    rows = _gather(weights, flat)
    return rows.reshape(_BATCH, _HIST, _DIM)


# speedup vs baseline: 1.6967x; 1.6415x over previous
import functools

import jax
import jax.numpy as jnp
from jax import lax
from jax.experimental import pallas as pl
from jax.experimental.pallas import tpu as pltpu
from jax.experimental.pallas import tpu_sc as plsc

_NUM_EMBEDDINGS = 1000000
_DIM = 64
_BATCH = 16384
_HIST = 200
_B = _BATCH * _HIST

_NC = 2
_NS = 16
_NW = _NC * _NS

_B_PER_W = _B // _NW
_CHUNK = 400
_ROWS_PER_CHUNK = _CHUNK // _HIST
_BROWS_PER_W = _BATCH // _NW
_STEPS = _B_PER_W // _CHUNK
_G = _STEPS // 2


def _body(table_hbm, idx_hbm, out_hbm, idx_v, rows_v,
          isem0, isem1, gsem0, gsem1, osem0, osem1):
    wid = lax.axis_index("s") * _NC + lax.axis_index("c")
    base = wid * _B_PER_W
    isems = (isem0, isem1)
    gsems = (gsem0, gsem1)
    osems = (osem0, osem1)

    def start_idx(i, b):
        off = base + i * _CHUNK
        pltpu.async_copy(idx_hbm.at[pl.ds(off, _CHUNK)], idx_v.at[b], isems[b])

    def wait_idx(b):
        pltpu.make_async_copy(idx_hbm.at[pl.ds(0, _CHUNK)], idx_v.at[b],
                              isems[b]).wait()

    def start_gather(b):
        for j in range(_ROWS_PER_CHUNK):
            pltpu.async_copy(
                table_hbm.at[idx_v.at[b, pl.ds(j * _HIST, _HIST)]],
                rows_v.at[b, j], gsems[b])

    def wait_gather(b):
        for j in range(_ROWS_PER_CHUNK):
            pltpu.make_async_copy(
                table_hbm.at[idx_v.at[b, pl.ds(j * _HIST, _HIST)]],
                rows_v.at[b, j], gsems[b]).wait()

    def start_write(i, b):
        boff = wid * _BROWS_PER_W + i * _ROWS_PER_CHUNK
        pltpu.async_copy(
            rows_v.at[b],
            out_hbm.at[pl.ds(boff, _ROWS_PER_CHUNK), :, pl.ds(0, _DIM)],
            osems[b])

    def wait_write(b):
        pltpu.make_async_copy(
            rows_v.at[b],
            out_hbm.at[pl.ds(0, _ROWS_PER_CHUNK), :, pl.ds(0, _DIM)],
            osems[b]).wait()

    def chunk_step(i, b):
        nb = 1 - b

        @pl.when(i > 0)
        def _():
            wait_gather(nb)
            start_write(i - 1, nb)

        @pl.when(i + 1 < _STEPS)
        def _():
            start_idx(i + 1, nb)

        wait_idx(b)

        @pl.when(i > 1)
        def _():
            wait_write(b)

        start_gather(b)

    start_idx(0, 0)

    def gstep(g, carry):
        i0 = 2 * g
        chunk_step(i0, 0)
        chunk_step(i0 + 1, 1)
        return carry

    lax.fori_loop(0, _G, gstep, 0)

    last = _STEPS - 1
    wait_gather(1)
    start_write(last, 1)
    wait_write(0)
    wait_write(1)


_gather = functools.partial(
    pl.kernel,
    out_type=jax.ShapeDtypeStruct((_BATCH, _HIST, 2 * _DIM), jnp.float32),
    mesh=plsc.VectorSubcoreMesh(core_axis_name="c", subcore_axis_name="s"),
    scratch_types=[
        pltpu.VMEM((2, _CHUNK), jnp.int32),
        pltpu.VMEM((2, _ROWS_PER_CHUNK, _HIST, _DIM), jnp.float32),
        pltpu.SemaphoreType.DMA,
        pltpu.SemaphoreType.DMA,
        pltpu.SemaphoreType.DMA,
        pltpu.SemaphoreType.DMA,
        pltpu.SemaphoreType.DMA,
        pltpu.SemaphoreType.DMA,
    ],
    compiler_params=pltpu.CompilerParams(use_tc_tiling_on_sc=False),
)(_body)


@jax.jit
def kernel(token_ids, weights):
    flat = token_ids.reshape(_B)
    padded = _gather(weights, flat)
    return padded[:, :, :_DIM]
